# baseline (speedup 1.0000x reference)
import functools

import jax
import jax.numpy as jnp
from jax import lax
from jax.experimental import pallas as pl
from jax.experimental.pallas import tpu as pltpu
from jax.experimental.pallas import tpu_sc as plsc

N = 10000
E = 320000
R = 8
D_IN = 128
H = 16

NC = 2
NS = 16
NW = NC * NS
ROWS = N * R
EW = E // NW
CHUNK = 400
NCHUNK = EW // CHUNK
TR = ROWS // NS
ZR = 250
ZCOPIES = TR // ZR
ONES_LEN = -(-CHUNK // 16) * 16
CZ_LEN = 1008


def _proj_idx_body(x_ref, w_ref, src_ref, dst_ref, et_ref,
                   t_ref, r_ref, g_ref, s_ref):
    res = jnp.dot(x_ref[...], w_ref[...], preferred_element_type=jnp.float32)
    t_ref[...] = res[:, : R * H]
    r_ref[...] = res[:, R * H :]
    et = et_ref[...]
    g_ref[...] = src_ref[...] * R + et
    s_ref[...] = dst_ref[...] * R + et


def _project_and_indices(x, wcat, src, dst, et):
    grid = 25
    bn = N // grid
    erows, ecols = E // (grid * 128), 128
    k = x.shape[1]
    wc = R * H + H
    eshape = (grid, erows, ecols)
    espec = pl.BlockSpec((1, erows, ecols), lambda i: (i, 0, 0))
    out = pl.pallas_call(
        _proj_idx_body,
        grid=(grid,),
        in_specs=[
            pl.BlockSpec((bn, k), lambda i: (i, 0)),
            pl.BlockSpec((k, wc), lambda i: (0, 0)),
            espec, espec, espec,
        ],
        out_specs=[
            pl.BlockSpec((bn, R * H), lambda i: (i, 0)),
            pl.BlockSpec((bn, H), lambda i: (i, 0)),
            espec, espec,
        ],
        out_shape=[
            jax.ShapeDtypeStruct((N, R * H), jnp.float32),
            jax.ShapeDtypeStruct((N, H), jnp.float32),
            jax.ShapeDtypeStruct(eshape, jnp.int32),
            jax.ShapeDtypeStruct(eshape, jnp.int32),
        ],
    )(x, wcat, src.reshape(eshape), dst.reshape(eshape),
      et.reshape(eshape))
    table, rootp, g2, s2 = out
    return table, rootp, g2.reshape(E), s2.reshape(E)


def _norm1_body(rootp_ref, agg_ref, cnt_ref, b_ref, w2_ref, t2_ref, r2_ref,
                inv_ref):
    cnt = cnt_ref[0] + cnt_ref[1]
    inv = 1.0 / jnp.maximum(cnt, 1.0)
    agg = agg_ref[0] + agg_ref[1]
    h = rootp_ref[...] + b_ref[0:1, :]
    for r in range(R):
        h = h + agg[:, r * H : (r + 1) * H] * inv[:, r : r + 1]
    h = jnp.maximum(h, 0.0)
    res = jnp.dot(h, w2_ref[...], preferred_element_type=jnp.float32)
    t2_ref[...] = res[:, : R * H]
    r2_ref[...] = res[:, R * H :]
    inv_ref[...] = inv


def _normalize_relu_project(rootp, agg, cnt, b1b, wcat2):
    bn = 2000
    wc = R * H + H
    return pl.pallas_call(
        _norm1_body,
        grid=(N // bn,),
        in_specs=[
            pl.BlockSpec((bn, H), lambda i: (i, 0)),
            pl.BlockSpec((NC, bn, R * H), lambda i: (0, i, 0)),
            pl.BlockSpec((NC, bn, R), lambda i: (0, i, 0)),
            pl.BlockSpec((8, H), lambda i: (0, 0)),
            pl.BlockSpec((H, wc), lambda i: (0, 0)),
        ],
        out_specs=[
            pl.BlockSpec((bn, R * H), lambda i: (i, 0)),
            pl.BlockSpec((bn, H), lambda i: (i, 0)),
            pl.BlockSpec((bn, R), lambda i: (i, 0)),
        ],
        out_shape=[
            jax.ShapeDtypeStruct((N, R * H), jnp.float32),
            jax.ShapeDtypeStruct((N, H), jnp.float32),
            jax.ShapeDtypeStruct((N, R), jnp.float32),
        ],
    )(rootp, agg, cnt, b1b, wcat2)


def _final_body(rootp_ref, agg_ref, inv_ref, b_ref, out_ref):
    agg = agg_ref[0] + agg_ref[1]
    o = rootp_ref[...] + b_ref[0:1, :]
    for r in range(R):
        o = o + agg[:, r * H : (r + 1) * H] * inv_ref[:, r : r + 1]
    m = jnp.max(o, axis=1, keepdims=True)
    s = jnp.log(jnp.sum(jnp.exp(o - m), axis=1, keepdims=True))
    out_ref[...] = o - m - s


def _normalize_logsoftmax(rootp, agg, inv, b2b):
    bn = 2000
    return pl.pallas_call(
        _final_body,
        grid=(N // bn,),
        in_specs=[
            pl.BlockSpec((bn, H), lambda i: (i, 0)),
            pl.BlockSpec((NC, bn, R * H), lambda i: (0, i, 0)),
            pl.BlockSpec((bn, R), lambda i: (i, 0)),
            pl.BlockSpec((8, H), lambda i: (0, 0)),
        ],
        out_specs=pl.BlockSpec((bn, H), lambda i: (i, 0)),
        out_shape=jax.ShapeDtypeStruct((N, H), jnp.float32),
    )(rootp, agg, inv, b2b)


def _edge_pass(table, gidx, sidx, with_counts):
    mesh = plsc.VectorSubcoreMesh(
        core_axis_name="c", subcore_axis_name="s",
        num_cores=NC, num_subcores=NS)

    out_type = [jax.ShapeDtypeStruct((NC, ROWS, H), jnp.float32)]
    scratch = [
        pltpu.VMEM((EW,), jnp.int32),
        pltpu.VMEM((EW,), jnp.int32),
        pltpu.VMEM((CHUNK, H), jnp.float32),
        pltpu.VMEM((CHUNK, H), jnp.float32),
        pltpu.VMEM((ZR, H), jnp.float32),
        pltpu.VMEM_SHARED((ROWS, H), jnp.float32),
        pltpu.SemaphoreType.DMA,
        pltpu.SemaphoreType.DMA,
        pltpu.SemaphoreType.DMA,
        pltpu.SemaphoreType.DMA,
        pltpu.SemaphoreType.DMA,
        pltpu.SemaphoreType.DMA,
    ]
    if with_counts:
        out_type.append(jax.ShapeDtypeStruct((NC, NS, TR), jnp.float32))
        scratch += [
            pltpu.VMEM((ONES_LEN,), jnp.float32),
            pltpu.VMEM((CZ_LEN,), jnp.float32),
            pltpu.VMEM_SHARED((ROWS,), jnp.float32),
        ]

    def body(table_h, gidx_h, sidx_h, agg_out, *rest):
        if with_counts:
            (cnt_out, gidx_all, sidx_all, rows0, rows1, zero_v, agg_sh,
             isem, zsem, gsem0, gsem1, ssem0, ssem1,
             ones_v, czero_v, cnt_sh) = rest
        else:
            (gidx_all, sidx_all, rows0, rows1, zero_v, agg_sh,
             isem, zsem, gsem0, gsem1, ssem0, ssem1) = rest
        rows = (rows0, rows1)
        gsem = (gsem0, gsem1)
        ssem = (ssem0, ssem1)
        cid = lax.axis_index("c")
        sid = lax.axis_index("s")
        wid = cid * NS + sid

        off = pl.multiple_of(wid * EW, 8)
        idx_d = [pltpu.async_copy(gidx_h.at[pl.ds(off, EW)], gidx_all, isem),
                 pltpu.async_copy(sidx_h.at[pl.ds(off, EW)], sidx_all, isem)]

        def zfill(i, _):
            zero_v[i, :] = jnp.zeros((H,), jnp.float32)
            return 0
        lax.fori_loop(0, ZR, zfill, 0)

        zero_d = [
            pltpu.async_copy(
                zero_v, agg_sh.at[pl.ds(sid * TR + j * ZR, ZR)], zsem)
            for j in range(ZCOPIES)
        ]
        if with_counts:
            def fill1(i, _):
                ones_v[pl.ds(i * 16, 16)] = jnp.ones((16,), jnp.float32)
                return 0
            lax.fori_loop(0, ONES_LEN // 16, fill1, 0)

            def fill0(i, _):
                czero_v[pl.ds(i * 16, 16)] = jnp.zeros((16,), jnp.float32)
                return 0
            lax.fori_loop(0, CZ_LEN // 16, fill0, 0)
            done = 0
            while done < TR:
                step = min(CZ_LEN, TR - done)
                zero_d.append(pltpu.async_copy(
                    czero_v.at[pl.ds(0, step)],
                    cnt_sh.at[pl.ds(sid * TR + done, step)], zsem))
                done += step

        for d in idx_d:
            d.wait()

        gather_d = {}
        scatter_d = {}

        def issue_gather(i):
            b = i % 2
            return pltpu.async_copy(
                table_h.at[gidx_all.at[pl.ds(i * CHUNK, CHUNK)]],
                rows[b], gsem[b])

        gather_d[0] = issue_gather(0)
        for d in zero_d:
            d.wait()
        plsc.subcore_barrier()

        for i in range(NCHUNK):
            b = i % 2
            if i + 1 < NCHUNK:
                if i - 1 >= 0:
                    for d in scatter_d.pop(i - 1):
                        d.wait()
                gather_d[i + 1] = issue_gather(i + 1)
            gather_d.pop(i).wait()
            sl = sidx_all.at[pl.ds(i * CHUNK, CHUNK)]
            ds_ = [pltpu.async_copy(rows[b], agg_sh.at[sl], ssem[b],
                                    add=True)]
            if with_counts:
                ds_.append(pltpu.async_copy(ones_v.at[pl.ds(0, CHUNK)],
                                            cnt_sh.at[sl], ssem[b],
                                            add=True))
            scatter_d[i] = ds_
        for k in sorted(scatter_d):
            for d in scatter_d[k]:
                d.wait()

        plsc.subcore_barrier()

        pltpu.sync_copy(agg_sh.at[pl.ds(sid * TR, TR)],
                        agg_out.at[cid, pl.ds(sid * TR, TR)])
        if with_counts:
            pltpu.sync_copy(cnt_sh.at[pl.ds(sid * TR, TR)],
                            cnt_out.at[cid, sid])

    run = pl.kernel(body, out_type=out_type, mesh=mesh,
                    scratch_types=scratch,
                    compiler_params=pltpu.CompilerParams(
                        use_tc_tiling_on_sc=False))
    return run(table, gidx, sidx)


@jax.jit
def _rgcn(x, edge_index, edge_type, W1, root1, b1, W2, root2, b2):
    src, dst = edge_index[0], edge_index[1]

    wcat1 = jnp.concatenate(
        [W1.transpose(1, 0, 2).reshape(D_IN, R * H), root1], axis=1)
    wcat2 = jnp.concatenate(
        [W2.transpose(1, 0, 2).reshape(H, R * H), root2], axis=1)
    b1b = jnp.broadcast_to(b1[None, :], (8, H))
    b2b = jnp.broadcast_to(b2[None, :], (8, H))

    table1, rootp1, gidx, sidx = _project_and_indices(
        x, wcat1, src, dst, edge_type)
    agg1, cnt = _edge_pass(table1.reshape(ROWS, H), gidx, sidx,
                           with_counts=True)

    table2, rootp2, inv = _normalize_relu_project(
        rootp1, agg1.reshape(NC, N, R * H), cnt.reshape(NC, N, R),
        b1b, wcat2)

    (agg2,) = _edge_pass(table2.reshape(ROWS, H), gidx, sidx,
                         with_counts=False)

    return _normalize_logsoftmax(rootp2, agg2.reshape(NC, N, R * H),
                                 inv, b2b)


def kernel(x, edge_index, edge_type, W1, root1, b1, W2, root2, b2):
    return _rgcn(x, edge_index, edge_type, W1, root1, b1, W2, root2, b2)


# speedup vs baseline: 1.0325x; 1.0325x over previous
import functools

import jax
import jax.numpy as jnp
from jax import lax
from jax.experimental import pallas as pl
from jax.experimental.pallas import tpu as pltpu
from jax.experimental.pallas import tpu_sc as plsc

N = 10000
E = 320000
R = 8
D_IN = 128
H = 16

NC = 2
NS = 16
NW = NC * NS
ROWS = N * R
EW = E // NW
CHUNK = 400
NCHUNK = EW // CHUNK
TR = ROWS // NS
ZR = 250
ZCOPIES = TR // ZR
ONES_LEN = -(-CHUNK // 16) * 16
CZ_LEN = 1008


def _idx_body(src_ref, dst_ref, et_ref, g_ref, s_ref):
    et = et_ref[...]
    g_ref[...] = src_ref[...] * R + et
    s_ref[...] = dst_ref[...] * R + et


def _edge_indices(src, dst, et):
    rows, cols = 625, 512
    src2 = src.reshape(rows, cols)
    dst2 = dst.reshape(rows, cols)
    et2 = et.reshape(rows, cols)
    g, s = pl.pallas_call(
        _idx_body,
        grid=(1,),
        in_specs=[pl.BlockSpec((rows, cols), lambda i: (0, 0))] * 3,
        out_specs=[pl.BlockSpec((rows, cols), lambda i: (0, 0))] * 2,
        out_shape=[jax.ShapeDtypeStruct((rows, cols), jnp.int32)] * 2,
    )(src2, dst2, et2)
    return g.reshape(E), s.reshape(E)


def _proj_body(x_ref, w_ref, t_ref, r_ref):
    res = jnp.dot(x_ref[...], w_ref[...], preferred_element_type=jnp.float32)
    t_ref[...] = res[:, : R * H]
    r_ref[...] = res[:, R * H :]


def _project(x, wcat):
    bn = 1000
    k = x.shape[1]
    wc = R * H + H
    return pl.pallas_call(
        _proj_body,
        grid=(N // bn,),
        in_specs=[
            pl.BlockSpec((bn, k), lambda i: (i, 0)),
            pl.BlockSpec((k, wc), lambda i: (0, 0)),
        ],
        out_specs=[
            pl.BlockSpec((bn, R * H), lambda i: (i, 0)),
            pl.BlockSpec((bn, H), lambda i: (i, 0)),
        ],
        out_shape=[
            jax.ShapeDtypeStruct((N, R * H), jnp.float32),
            jax.ShapeDtypeStruct((N, H), jnp.float32),
        ],
    )(x, wcat)


def _merge_msgs(rootp, agg, cnt, b):
    inv = 1.0 / jnp.maximum(cnt[0] + cnt[1], 1.0)
    a = agg[0] + agg[1]
    o = rootp + b[0:1, :]
    for r in range(R):
        o = o + a[:, r * H : (r + 1) * H] * inv[:, r : r + 1]
    return o


def _norm1_body(rootp_ref, agg_ref, cnt_ref, b_ref, w2_ref, t2_ref, r2_ref):
    h = _merge_msgs(rootp_ref[...], agg_ref[...], cnt_ref[...], b_ref[...])
    h = jnp.maximum(h, 0.0)
    res = jnp.dot(h, w2_ref[...], preferred_element_type=jnp.float32)
    t2_ref[...] = res[:, : R * H]
    r2_ref[...] = res[:, R * H :]


def _normalize_relu_project(rootp, agg, cnt, b1b, wcat2):
    bn = 2000
    wc = R * H + H
    return pl.pallas_call(
        _norm1_body,
        grid=(N // bn,),
        in_specs=[
            pl.BlockSpec((bn, H), lambda i: (i, 0)),
            pl.BlockSpec((NC, bn, R * H), lambda i: (0, i, 0)),
            pl.BlockSpec((NC, bn, R), lambda i: (0, i, 0)),
            pl.BlockSpec((8, H), lambda i: (0, 0)),
            pl.BlockSpec((H, wc), lambda i: (0, 0)),
        ],
        out_specs=[
            pl.BlockSpec((bn, R * H), lambda i: (i, 0)),
            pl.BlockSpec((bn, H), lambda i: (i, 0)),
        ],
        out_shape=[
            jax.ShapeDtypeStruct((N, R * H), jnp.float32),
            jax.ShapeDtypeStruct((N, H), jnp.float32),
        ],
    )(rootp, agg, cnt, b1b, wcat2)


def _final_body(rootp_ref, agg_ref, cnt_ref, b_ref, out_ref):
    o = _merge_msgs(rootp_ref[...], agg_ref[...], cnt_ref[...], b_ref[...])
    m = jnp.max(o, axis=1, keepdims=True)
    s = jnp.log(jnp.sum(jnp.exp(o - m), axis=1, keepdims=True))
    out_ref[...] = o - m - s


def _normalize_logsoftmax(rootp, agg, cnt, b2b):
    bn = 2000
    return pl.pallas_call(
        _final_body,
        grid=(N // bn,),
        in_specs=[
            pl.BlockSpec((bn, H), lambda i: (i, 0)),
            pl.BlockSpec((NC, bn, R * H), lambda i: (0, i, 0)),
            pl.BlockSpec((NC, bn, R), lambda i: (0, i, 0)),
            pl.BlockSpec((8, H), lambda i: (0, 0)),
        ],
        out_specs=pl.BlockSpec((bn, H), lambda i: (i, 0)),
        out_shape=jax.ShapeDtypeStruct((N, H), jnp.float32),
    )(rootp, agg, cnt, b2b)


def _edge_pass(table, gidx, sidx, with_counts):
    mesh = plsc.VectorSubcoreMesh(
        core_axis_name="c", subcore_axis_name="s",
        num_cores=NC, num_subcores=NS)

    out_type = [jax.ShapeDtypeStruct((NC, ROWS, H), jnp.float32)]
    scratch = [
        pltpu.VMEM((EW,), jnp.int32),
        pltpu.VMEM((EW,), jnp.int32),
        pltpu.VMEM((CHUNK, H), jnp.float32),
        pltpu.VMEM((CHUNK, H), jnp.float32),
        pltpu.VMEM((ZR, H), jnp.float32),
        pltpu.VMEM_SHARED((ROWS, H), jnp.float32),
        pltpu.SemaphoreType.DMA,
        pltpu.SemaphoreType.DMA,
        pltpu.SemaphoreType.DMA,
        pltpu.SemaphoreType.DMA,
        pltpu.SemaphoreType.DMA,
        pltpu.SemaphoreType.DMA,
    ]
    if with_counts:
        out_type.append(jax.ShapeDtypeStruct((NC, NS, TR), jnp.float32))
        scratch += [
            pltpu.VMEM((ONES_LEN,), jnp.float32),
            pltpu.VMEM((CZ_LEN,), jnp.float32),
            pltpu.VMEM_SHARED((ROWS,), jnp.float32),
        ]

    def body(table_h, gidx_h, sidx_h, agg_out, *rest):
        if with_counts:
            (cnt_out, gidx_all, sidx_all, rows0, rows1, zero_v, agg_sh,
             isem, zsem, gsem0, gsem1, ssem0, ssem1,
             ones_v, czero_v, cnt_sh) = rest
        else:
            (gidx_all, sidx_all, rows0, rows1, zero_v, agg_sh,
             isem, zsem, gsem0, gsem1, ssem0, ssem1) = rest
        rows = (rows0, rows1)
        gsem = (gsem0, gsem1)
        ssem = (ssem0, ssem1)
        cid = lax.axis_index("c")
        sid = lax.axis_index("s")
        wid = cid * NS + sid

        off = pl.multiple_of(wid * EW, 8)
        idx_d = [pltpu.async_copy(gidx_h.at[pl.ds(off, EW)], gidx_all, isem),
                 pltpu.async_copy(sidx_h.at[pl.ds(off, EW)], sidx_all, isem)]

        def zfill(i, _):
            zero_v[i, :] = jnp.zeros((H,), jnp.float32)
            return 0
        lax.fori_loop(0, ZR, zfill, 0)

        zero_d = [
            pltpu.async_copy(
                zero_v, agg_sh.at[pl.ds(sid * TR + j * ZR, ZR)], zsem)
            for j in range(ZCOPIES)
        ]
        if with_counts:
            def fill1(i, _):
                ones_v[pl.ds(i * 16, 16)] = jnp.ones((16,), jnp.float32)
                return 0
            lax.fori_loop(0, ONES_LEN // 16, fill1, 0)

            def fill0(i, _):
                czero_v[pl.ds(i * 16, 16)] = jnp.zeros((16,), jnp.float32)
                return 0
            lax.fori_loop(0, CZ_LEN // 16, fill0, 0)
            done = 0
            while done < TR:
                step = min(CZ_LEN, TR - done)
                zero_d.append(pltpu.async_copy(
                    czero_v.at[pl.ds(0, step)],
                    cnt_sh.at[pl.ds(sid * TR + done, step)], zsem))
                done += step

        for d in idx_d:
            d.wait()

        gather_d = {}
        scatter_d = {}

        def issue_gather(i):
            b = i % 2
            return pltpu.async_copy(
                table_h.at[gidx_all.at[pl.ds(i * CHUNK, CHUNK)]],
                rows[b], gsem[b])

        gather_d[0] = issue_gather(0)
        for d in zero_d:
            d.wait()
        plsc.subcore_barrier()

        for i in range(NCHUNK):
            b = i % 2
            if i + 1 < NCHUNK:
                if i - 1 >= 0:
                    for d in scatter_d.pop(i - 1):
                        d.wait()
                gather_d[i + 1] = issue_gather(i + 1)
            gather_d.pop(i).wait()
            sl = sidx_all.at[pl.ds(i * CHUNK, CHUNK)]
            ds_ = [pltpu.async_copy(rows[b], agg_sh.at[sl], ssem[b],
                                    add=True)]
            if with_counts:
                ds_.append(pltpu.async_copy(ones_v.at[pl.ds(0, CHUNK)],
                                            cnt_sh.at[sl], ssem[b],
                                            add=True))
            scatter_d[i] = ds_
        for k in sorted(scatter_d):
            for d in scatter_d[k]:
                d.wait()

        plsc.subcore_barrier()

        pltpu.sync_copy(agg_sh.at[pl.ds(sid * TR, TR)],
                        agg_out.at[cid, pl.ds(sid * TR, TR)])
        if with_counts:
            pltpu.sync_copy(cnt_sh.at[pl.ds(sid * TR, TR)],
                            cnt_out.at[cid, sid])

    run = pl.kernel(body, out_type=out_type, mesh=mesh,
                    scratch_types=scratch,
                    compiler_params=pltpu.CompilerParams(
                        use_tc_tiling_on_sc=False))
    return run(table, gidx, sidx)


@jax.jit
def _rgcn(x, edge_index, edge_type, W1, root1, b1, W2, root2, b2):
    src, dst = edge_index[0], edge_index[1]

    wcat1 = jnp.concatenate(
        [W1.transpose(1, 0, 2).reshape(D_IN, R * H), root1], axis=1)
    wcat2 = jnp.concatenate(
        [W2.transpose(1, 0, 2).reshape(H, R * H), root2], axis=1)
    b1b = jnp.broadcast_to(b1[None, :], (8, H))
    b2b = jnp.broadcast_to(b2[None, :], (8, H))

    gidx, sidx = _edge_indices(src, dst, edge_type)
    table1, rootp1 = _project(x, wcat1)
    agg1, cnt = _edge_pass(table1.reshape(ROWS, H), gidx, sidx,
                           with_counts=True)
    cnt_r = cnt.reshape(NC, N, R)

    table2, rootp2 = _normalize_relu_project(
        rootp1, agg1.reshape(NC, N, R * H), cnt_r, b1b, wcat2)

    (agg2,) = _edge_pass(table2.reshape(ROWS, H), gidx, sidx,
                         with_counts=False)

    return _normalize_logsoftmax(rootp2, agg2.reshape(NC, N, R * H),
                                 cnt_r, b2b)


def kernel(x, edge_index, edge_type, W1, root1, b1, W2, root2, b2):
    return _rgcn(x, edge_index, edge_type, W1, root1, b1, W2, root2, b2)


# speedup vs baseline: 1.0547x; 1.0215x over previous
import functools

import jax
import jax.numpy as jnp
from jax import lax
from jax.experimental import pallas as pl
from jax.experimental.pallas import tpu as pltpu
from jax.experimental.pallas import tpu_sc as plsc

N = 10000
E = 320000
R = 8
D_IN = 128
H = 16

NC = 2
NS = 16
NW = NC * NS
ROWS = N * R
EW = E // NW
CHUNK1 = 400
CHUNK2 = 840
TR = ROWS // NS
ZR = 250
ZCOPIES = TR // ZR
CZ_LEN = 1008


def _idx_body(src_ref, dst_ref, et_ref, g_ref, s_ref):
    et = et_ref[...]
    g_ref[...] = src_ref[...] * R + et
    s_ref[...] = dst_ref[...] * R + et


def _edge_indices(src, dst, et):
    rows, cols = 625, 512
    src2 = src.reshape(rows, cols)
    dst2 = dst.reshape(rows, cols)
    et2 = et.reshape(rows, cols)
    g, s = pl.pallas_call(
        _idx_body,
        grid=(1,),
        in_specs=[pl.BlockSpec((rows, cols), lambda i: (0, 0))] * 3,
        out_specs=[pl.BlockSpec((rows, cols), lambda i: (0, 0))] * 2,
        out_shape=[jax.ShapeDtypeStruct((rows, cols), jnp.int32)] * 2,
    )(src2, dst2, et2)
    return g.reshape(E), s.reshape(E)


def _proj_body(x_ref, w_ref, t_ref, r_ref):
    res = jnp.dot(x_ref[...], w_ref[...], preferred_element_type=jnp.float32)
    t_ref[...] = res[:, : R * H]
    r_ref[...] = res[:, R * H :]


def _project(x, wcat):
    bn = 1000
    k = x.shape[1]
    wc = R * H + H
    return pl.pallas_call(
        _proj_body,
        grid=(N // bn,),
        in_specs=[
            pl.BlockSpec((bn, k), lambda i: (i, 0)),
            pl.BlockSpec((k, wc), lambda i: (0, 0)),
        ],
        out_specs=[
            pl.BlockSpec((bn, R * H), lambda i: (i, 0)),
            pl.BlockSpec((bn, H), lambda i: (i, 0)),
        ],
        out_shape=[
            jax.ShapeDtypeStruct((N, R * H), jnp.float32),
            jax.ShapeDtypeStruct((N, H), jnp.float32),
        ],
    )(x, wcat)


def _merge_msgs(rootp, agg, cnt, b):
    inv = 1.0 / jnp.maximum(cnt[0] + cnt[1], 1.0)
    a = agg[0] + agg[1]
    o = rootp + b[0:1, :]
    for r in range(R):
        o = o + a[:, r * H : (r + 1) * H] * inv[:, r : r + 1]
    return o


def _norm1_body(rootp_ref, agg_ref, cnt_ref, b_ref, w2_ref, t2_ref, r2_ref):
    h = _merge_msgs(rootp_ref[...], agg_ref[...], cnt_ref[...], b_ref[...])
    h = jnp.maximum(h, 0.0)
    res = jnp.dot(h, w2_ref[...], preferred_element_type=jnp.float32)
    t2_ref[...] = res[:, : R * H]
    r2_ref[...] = res[:, R * H :]


def _normalize_relu_project(rootp, agg, cnt, b1b, wcat2):
    bn = 2000
    wc = R * H + H
    return pl.pallas_call(
        _norm1_body,
        grid=(N // bn,),
        in_specs=[
            pl.BlockSpec((bn, H), lambda i: (i, 0)),
            pl.BlockSpec((NC, bn, R * H), lambda i: (0, i, 0)),
            pl.BlockSpec((NC, bn, R), lambda i: (0, i, 0)),
            pl.BlockSpec((8, H), lambda i: (0, 0)),
            pl.BlockSpec((H, wc), lambda i: (0, 0)),
        ],
        out_specs=[
            pl.BlockSpec((bn, R * H), lambda i: (i, 0)),
            pl.BlockSpec((bn, H), lambda i: (i, 0)),
        ],
        out_shape=[
            jax.ShapeDtypeStruct((N, R * H), jnp.float32),
            jax.ShapeDtypeStruct((N, H), jnp.float32),
        ],
    )(rootp, agg, cnt, b1b, wcat2)


def _final_body(rootp_ref, agg_ref, cnt_ref, b_ref, out_ref):
    o = _merge_msgs(rootp_ref[...], agg_ref[...], cnt_ref[...], b_ref[...])
    m = jnp.max(o, axis=1, keepdims=True)
    s = jnp.log(jnp.sum(jnp.exp(o - m), axis=1, keepdims=True))
    out_ref[...] = o - m - s


def _normalize_logsoftmax(rootp, agg, cnt, b2b):
    bn = 2000
    return pl.pallas_call(
        _final_body,
        grid=(N // bn,),
        in_specs=[
            pl.BlockSpec((bn, H), lambda i: (i, 0)),
            pl.BlockSpec((NC, bn, R * H), lambda i: (0, i, 0)),
            pl.BlockSpec((NC, bn, R), lambda i: (0, i, 0)),
            pl.BlockSpec((8, H), lambda i: (0, 0)),
        ],
        out_specs=pl.BlockSpec((bn, H), lambda i: (i, 0)),
        out_shape=jax.ShapeDtypeStruct((N, H), jnp.float32),
    )(rootp, agg, cnt, b2b)


def _edge_pass(table, gidx, sidx, with_counts, chunk):
    nfull = EW // chunk
    tail = EW - nfull * chunk
    sizes = [chunk] * nfull + ([tail] if tail else [])
    nchunk = len(sizes)
    ones_len = -(-chunk // 16) * 16
    mesh = plsc.VectorSubcoreMesh(
        core_axis_name="c", subcore_axis_name="s",
        num_cores=NC, num_subcores=NS)

    out_type = [jax.ShapeDtypeStruct((NC, ROWS, H), jnp.float32)]
    scratch = [
        pltpu.VMEM((EW,), jnp.int32),
        pltpu.VMEM((EW,), jnp.int32),
        pltpu.VMEM((chunk, H), jnp.float32),
        pltpu.VMEM((chunk, H), jnp.float32),
        pltpu.VMEM((ZR, H), jnp.float32),
        pltpu.VMEM_SHARED((ROWS, H), jnp.float32),
        pltpu.SemaphoreType.DMA,
        pltpu.SemaphoreType.DMA,
        pltpu.SemaphoreType.DMA,
        pltpu.SemaphoreType.DMA,
        pltpu.SemaphoreType.DMA,
        pltpu.SemaphoreType.DMA,
    ]
    if with_counts:
        out_type.append(jax.ShapeDtypeStruct((NC, NS, TR), jnp.float32))
        scratch += [
            pltpu.VMEM((ones_len,), jnp.float32),
            pltpu.VMEM((CZ_LEN,), jnp.float32),
            pltpu.VMEM_SHARED((ROWS,), jnp.float32),
        ]

    def body(table_h, gidx_h, sidx_h, agg_out, *rest):
        if with_counts:
            (cnt_out, gidx_all, sidx_all, rows0, rows1, zero_v, agg_sh,
             isem, zsem, gsem0, gsem1, ssem0, ssem1,
             ones_v, czero_v, cnt_sh) = rest
        else:
            (gidx_all, sidx_all, rows0, rows1, zero_v, agg_sh,
             isem, zsem, gsem0, gsem1, ssem0, ssem1) = rest
        rows = (rows0, rows1)
        gsem = (gsem0, gsem1)
        ssem = (ssem0, ssem1)
        cid = lax.axis_index("c")
        sid = lax.axis_index("s")
        wid = cid * NS + sid

        off = pl.multiple_of(wid * EW, 8)
        idx_d = [pltpu.async_copy(gidx_h.at[pl.ds(off, EW)], gidx_all, isem),
                 pltpu.async_copy(sidx_h.at[pl.ds(off, EW)], sidx_all, isem)]

        def zfill(i, _):
            zero_v[i, :] = jnp.zeros((H,), jnp.float32)
            return 0
        lax.fori_loop(0, ZR, zfill, 0)

        zero_d = [
            pltpu.async_copy(
                zero_v, agg_sh.at[pl.ds(sid * TR + j * ZR, ZR)], zsem)
            for j in range(ZCOPIES)
        ]
        if with_counts:
            def fill1(i, _):
                ones_v[pl.ds(i * 16, 16)] = jnp.ones((16,), jnp.float32)
                return 0
            lax.fori_loop(0, ones_len // 16, fill1, 0)

            def fill0(i, _):
                czero_v[pl.ds(i * 16, 16)] = jnp.zeros((16,), jnp.float32)
                return 0
            lax.fori_loop(0, CZ_LEN // 16, fill0, 0)
            done = 0
            while done < TR:
                step = min(CZ_LEN, TR - done)
                zero_d.append(pltpu.async_copy(
                    czero_v.at[pl.ds(0, step)],
                    cnt_sh.at[pl.ds(sid * TR + done, step)], zsem))
                done += step

        for d in idx_d:
            d.wait()

        gather_d = {}
        scatter_d = {}

        def issue_gather(i):
            b = i % 2
            return pltpu.async_copy(
                table_h.at[gidx_all.at[pl.ds(i * chunk, sizes[i])]],
                rows[b].at[pl.ds(0, sizes[i])], gsem[b])

        gather_d[0] = issue_gather(0)
        for d in zero_d:
            d.wait()
        plsc.subcore_barrier()

        for i in range(nchunk):
            b = i % 2
            if i + 1 < nchunk:
                if i - 1 >= 0:
                    for d in scatter_d.pop(i - 1):
                        d.wait()
                gather_d[i + 1] = issue_gather(i + 1)
            gather_d.pop(i).wait()
            sl = sidx_all.at[pl.ds(i * chunk, sizes[i])]
            ds_ = [pltpu.async_copy(rows[b].at[pl.ds(0, sizes[i])],
                                    agg_sh.at[sl], ssem[b], add=True)]
            if with_counts:
                ds_.append(pltpu.async_copy(ones_v.at[pl.ds(0, sizes[i])],
                                            cnt_sh.at[sl], ssem[b],
                                            add=True))
            scatter_d[i] = ds_
        for k in sorted(scatter_d):
            for d in scatter_d[k]:
                d.wait()

        plsc.subcore_barrier()

        pltpu.sync_copy(agg_sh.at[pl.ds(sid * TR, TR)],
                        agg_out.at[cid, pl.ds(sid * TR, TR)])
        if with_counts:
            pltpu.sync_copy(cnt_sh.at[pl.ds(sid * TR, TR)],
                            cnt_out.at[cid, sid])

    run = pl.kernel(body, out_type=out_type, mesh=mesh,
                    scratch_types=scratch,
                    compiler_params=pltpu.CompilerParams(
                        use_tc_tiling_on_sc=False))
    return run(table, gidx, sidx)


@jax.jit
def _rgcn(x, edge_index, edge_type, W1, root1, b1, W2, root2, b2):
    src, dst = edge_index[0], edge_index[1]

    wcat1 = jnp.concatenate(
        [W1.transpose(1, 0, 2).reshape(D_IN, R * H), root1], axis=1)
    wcat2 = jnp.concatenate(
        [W2.transpose(1, 0, 2).reshape(H, R * H), root2], axis=1)
    b1b = jnp.broadcast_to(b1[None, :], (8, H))
    b2b = jnp.broadcast_to(b2[None, :], (8, H))

    gidx, sidx = _edge_indices(src, dst, edge_type)
    table1, rootp1 = _project(x, wcat1)
    agg1, cnt = _edge_pass(table1.reshape(ROWS, H), gidx, sidx,
                           with_counts=True, chunk=CHUNK1)
    cnt_r = cnt.reshape(NC, N, R)

    table2, rootp2 = _normalize_relu_project(
        rootp1, agg1.reshape(NC, N, R * H), cnt_r, b1b, wcat2)

    (agg2,) = _edge_pass(table2.reshape(ROWS, H), gidx, sidx,
                         with_counts=False, chunk=CHUNK2)

    return _normalize_logsoftmax(rootp2, agg2.reshape(NC, N, R * H),
                                 cnt_r, b2b)


def kernel(x, edge_index, edge_type, W1, root1, b1, W2, root2, b2):
    return _rgcn(x, edge_index, edge_type, W1, root1, b1, W2, root2, b2)


# speedup vs baseline: 1.0721x; 1.0165x over previous
import functools

import jax
import jax.numpy as jnp
from jax import lax
from jax.experimental import pallas as pl
from jax.experimental.pallas import tpu as pltpu
from jax.experimental.pallas import tpu_sc as plsc

N = 10000
E = 320000
R = 8
D_IN = 128
H = 16

NC = 2
NS = 16
NW = NC * NS
ROWS = N * R
EW = E // NW
CHUNK1 = 624
CHUNK2 = 840
TR = ROWS // NS
ZR = 250
ZCOPIES = TR // ZR
CZ_LEN = 1008


def _idx_body(src_ref, dst_ref, et_ref, g_ref, s_ref):
    et = et_ref[...]
    g_ref[...] = src_ref[...] * R + et
    s_ref[...] = dst_ref[...] * R + et


def _edge_indices(src, dst, et):
    rows, cols = 625, 512
    src2 = src.reshape(rows, cols)
    dst2 = dst.reshape(rows, cols)
    et2 = et.reshape(rows, cols)
    g, s = pl.pallas_call(
        _idx_body,
        grid=(1,),
        in_specs=[pl.BlockSpec((rows, cols), lambda i: (0, 0))] * 3,
        out_specs=[pl.BlockSpec((rows, cols), lambda i: (0, 0))] * 2,
        out_shape=[jax.ShapeDtypeStruct((rows, cols), jnp.int32)] * 2,
    )(src2, dst2, et2)
    return g.reshape(E), s.reshape(E)


def _proj_body(x_ref, w_ref, t_ref, r_ref):
    res = jnp.dot(x_ref[...], w_ref[...], preferred_element_type=jnp.float32)
    t_ref[...] = res[:, : R * H]
    r_ref[...] = res[:, R * H :]


def _project(x, wcat):
    bn = 1000
    k = x.shape[1]
    wc = R * H + H
    return pl.pallas_call(
        _proj_body,
        grid=(N // bn,),
        in_specs=[
            pl.BlockSpec((bn, k), lambda i: (i, 0)),
            pl.BlockSpec((k, wc), lambda i: (0, 0)),
        ],
        out_specs=[
            pl.BlockSpec((bn, R * H), lambda i: (i, 0)),
            pl.BlockSpec((bn, H), lambda i: (i, 0)),
        ],
        out_shape=[
            jax.ShapeDtypeStruct((N, R * H), jnp.float32),
            jax.ShapeDtypeStruct((N, H), jnp.float32),
        ],
    )(x, wcat)


def _merge_msgs(rootp, agg, cnt, b):
    inv = 1.0 / jnp.maximum(cnt[0] + cnt[1], 1.0)
    a = agg[0] + agg[1]
    o = rootp + b[0:1, :]
    for r in range(R):
        o = o + a[:, r * H : (r + 1) * H] * inv[:, r : r + 1]
    return o


def _norm1_body(rootp_ref, agg_ref, cnt_ref, b_ref, w2_ref, t2_ref, r2_ref):
    h = _merge_msgs(rootp_ref[...], agg_ref[...], cnt_ref[...], b_ref[...])
    h = jnp.maximum(h, 0.0)
    res = jnp.dot(h, w2_ref[...], preferred_element_type=jnp.float32)
    t2_ref[...] = res[:, : R * H]
    r2_ref[...] = res[:, R * H :]


def _normalize_relu_project(rootp, agg, cnt, b1b, wcat2):
    bn = 2000
    wc = R * H + H
    return pl.pallas_call(
        _norm1_body,
        grid=(N // bn,),
        in_specs=[
            pl.BlockSpec((bn, H), lambda i: (i, 0)),
            pl.BlockSpec((NC, bn, R * H), lambda i: (0, i, 0)),
            pl.BlockSpec((NC, bn, R), lambda i: (0, i, 0)),
            pl.BlockSpec((8, H), lambda i: (0, 0)),
            pl.BlockSpec((H, wc), lambda i: (0, 0)),
        ],
        out_specs=[
            pl.BlockSpec((bn, R * H), lambda i: (i, 0)),
            pl.BlockSpec((bn, H), lambda i: (i, 0)),
        ],
        out_shape=[
            jax.ShapeDtypeStruct((N, R * H), jnp.float32),
            jax.ShapeDtypeStruct((N, H), jnp.float32),
        ],
    )(rootp, agg, cnt, b1b, wcat2)


def _final_body(rootp_ref, agg_ref, cnt_ref, b_ref, out_ref):
    o = _merge_msgs(rootp_ref[...], agg_ref[...], cnt_ref[...], b_ref[...])
    m = jnp.max(o, axis=1, keepdims=True)
    s = jnp.log(jnp.sum(jnp.exp(o - m), axis=1, keepdims=True))
    out_ref[...] = o - m - s


def _normalize_logsoftmax(rootp, agg, cnt, b2b):
    bn = 2000
    return pl.pallas_call(
        _final_body,
        grid=(N // bn,),
        in_specs=[
            pl.BlockSpec((bn, H), lambda i: (i, 0)),
            pl.BlockSpec((NC, bn, R * H), lambda i: (0, i, 0)),
            pl.BlockSpec((NC, bn, R), lambda i: (0, i, 0)),
            pl.BlockSpec((8, H), lambda i: (0, 0)),
        ],
        out_specs=pl.BlockSpec((bn, H), lambda i: (i, 0)),
        out_shape=jax.ShapeDtypeStruct((N, H), jnp.float32),
    )(rootp, agg, cnt, b2b)


def _edge_pass(table, gidx, sidx, with_counts, chunk):
    nfull = EW // chunk
    tail = EW - nfull * chunk
    sizes = [chunk] * nfull + ([tail] if tail else [])
    nchunk = len(sizes)
    ones_len = -(-chunk // 16) * 16
    mesh = plsc.VectorSubcoreMesh(
        core_axis_name="c", subcore_axis_name="s",
        num_cores=NC, num_subcores=NS)

    out_type = [jax.ShapeDtypeStruct((NC, ROWS, H), jnp.float32)]
    scratch = [
        pltpu.VMEM((EW,), jnp.int32),
        pltpu.VMEM((EW,), jnp.int32),
        pltpu.VMEM((chunk, H), jnp.float32),
        pltpu.VMEM((chunk, H), jnp.float32),
        pltpu.VMEM((ZR, H), jnp.float32),
        pltpu.VMEM_SHARED((ROWS, H), jnp.float32),
        pltpu.SemaphoreType.DMA,
        pltpu.SemaphoreType.DMA,
        pltpu.SemaphoreType.DMA,
        pltpu.SemaphoreType.DMA,
        pltpu.SemaphoreType.DMA,
        pltpu.SemaphoreType.DMA,
    ]
    if with_counts:
        out_type.append(jax.ShapeDtypeStruct((NC, NS, TR), jnp.float32))
        scratch += [
            pltpu.VMEM((ones_len,), jnp.float32),
            pltpu.VMEM((CZ_LEN,), jnp.float32),
            pltpu.VMEM_SHARED((ROWS,), jnp.float32),
        ]

    def body(table_h, gidx_h, sidx_h, agg_out, *rest):
        if with_counts:
            (cnt_out, gidx_all, sidx_all, rows0, rows1, zero_v, agg_sh,
             isem, zsem, gsem0, gsem1, ssem0, ssem1,
             ones_v, czero_v, cnt_sh) = rest
        else:
            (gidx_all, sidx_all, rows0, rows1, zero_v, agg_sh,
             isem, zsem, gsem0, gsem1, ssem0, ssem1) = rest
        rows = (rows0, rows1)
        gsem = (gsem0, gsem1)
        ssem = (ssem0, ssem1)
        cid = lax.axis_index("c")
        sid = lax.axis_index("s")
        wid = cid * NS + sid

        off = pl.multiple_of(wid * EW, 8)
        idx_d = [pltpu.async_copy(gidx_h.at[pl.ds(off, EW)], gidx_all, isem),
                 pltpu.async_copy(sidx_h.at[pl.ds(off, EW)], sidx_all, isem)]

        def zfill(i, _):
            zero_v[i, :] = jnp.zeros((H,), jnp.float32)
            return 0
        lax.fori_loop(0, ZR, zfill, 0)

        zero_d = [
            pltpu.async_copy(
                zero_v, agg_sh.at[pl.ds(sid * TR + j * ZR, ZR)], zsem)
            for j in range(ZCOPIES)
        ]
        if with_counts:
            def fill1(i, _):
                ones_v[pl.ds(i * 16, 16)] = jnp.ones((16,), jnp.float32)
                return 0
            lax.fori_loop(0, ones_len // 16, fill1, 0)

            def fill0(i, _):
                czero_v[pl.ds(i * 16, 16)] = jnp.zeros((16,), jnp.float32)
                return 0
            lax.fori_loop(0, CZ_LEN // 16, fill0, 0)
            done = 0
            while done < TR:
                step = min(CZ_LEN, TR - done)
                zero_d.append(pltpu.async_copy(
                    czero_v.at[pl.ds(0, step)],
                    cnt_sh.at[pl.ds(sid * TR + done, step)], zsem))
                done += step

        for d in idx_d:
            d.wait()

        gather_d = {}
        scatter_d = {}

        def issue_gather(i):
            b = i % 2
            return pltpu.async_copy(
                table_h.at[gidx_all.at[pl.ds(i * chunk, sizes[i])]],
                rows[b].at[pl.ds(0, sizes[i])], gsem[b])

        gather_d[0] = issue_gather(0)
        for d in zero_d:
            d.wait()
        plsc.subcore_barrier()

        for i in range(nchunk):
            b = i % 2
            if i + 1 < nchunk:
                if i - 1 >= 0:
                    for d in scatter_d.pop(i - 1):
                        d.wait()
                gather_d[i + 1] = issue_gather(i + 1)
            gather_d.pop(i).wait()
            sl = sidx_all.at[pl.ds(i * chunk, sizes[i])]
            ds_ = [pltpu.async_copy(rows[b].at[pl.ds(0, sizes[i])],
                                    agg_sh.at[sl], ssem[b], add=True)]
            if with_counts:
                ds_.append(pltpu.async_copy(ones_v.at[pl.ds(0, sizes[i])],
                                            cnt_sh.at[sl], ssem[b],
                                            add=True))
            scatter_d[i] = ds_
        for k in sorted(scatter_d):
            for d in scatter_d[k]:
                d.wait()

        plsc.subcore_barrier()

        pltpu.sync_copy(agg_sh.at[pl.ds(sid * TR, TR)],
                        agg_out.at[cid, pl.ds(sid * TR, TR)])
        if with_counts:
            pltpu.sync_copy(cnt_sh.at[pl.ds(sid * TR, TR)],
                            cnt_out.at[cid, sid])

    run = pl.kernel(body, out_type=out_type, mesh=mesh,
                    scratch_types=scratch,
                    compiler_params=pltpu.CompilerParams(
                        use_tc_tiling_on_sc=False))
    return run(table, gidx, sidx)


@jax.jit
def _rgcn(x, edge_index, edge_type, W1, root1, b1, W2, root2, b2):
    src, dst = edge_index[0], edge_index[1]

    wcat1 = jnp.concatenate(
        [W1.transpose(1, 0, 2).reshape(D_IN, R * H), root1], axis=1)
    wcat2 = jnp.concatenate(
        [W2.transpose(1, 0, 2).reshape(H, R * H), root2], axis=1)
    b1b = jnp.broadcast_to(b1[None, :], (8, H))
    b2b = jnp.broadcast_to(b2[None, :], (8, H))

    gidx, sidx = _edge_indices(src, dst, edge_type)
    table1, rootp1 = _project(x, wcat1)
    agg1, cnt = _edge_pass(table1.reshape(ROWS, H), gidx, sidx,
                           with_counts=True, chunk=CHUNK1)
    cnt_r = cnt.reshape(NC, N, R)

    table2, rootp2 = _normalize_relu_project(
        rootp1, agg1.reshape(NC, N, R * H), cnt_r, b1b, wcat2)

    (agg2,) = _edge_pass(table2.reshape(ROWS, H), gidx, sidx,
                         with_counts=False, chunk=CHUNK2)

    return _normalize_logsoftmax(rootp2, agg2.reshape(NC, N, R * H),
                                 cnt_r, b2b)


def kernel(x, edge_index, edge_type, W1, root1, b1, W2, root2, b2):
    return _rgcn(x, edge_index, edge_type, W1, root1, b1, W2, root2, b2)
